# baseline (speedup 1.0000x reference)
import jax
import jax.numpy as jnp
from jax import lax
from jax.experimental import pallas as pl
from jax.experimental.pallas import tpu as pltpu
from jax.experimental.pallas import tpu_sc as plsc

NC = 2
NS = 16
LANES = 16
C = 128
Q0 = 56
Q1 = 104


def _sc_segment_sum(n_feat, src_r, dst_r, w_r):
    N, D = n_feat.shape
    rpt = -(-N // (NS * 8)) * 8
    Npad = rpt * NS
    n_full = rpt // C
    tail = rpt - n_full * C
    mesh = plsc.VectorSubcoreMesh(
        core_axis_name="c", subcore_axis_name="s",
        num_cores=NC, num_subcores=NS)

    PC = 8

    def body(nfeat_hbm, src_hbm, dst_hbm, w_hbm, out_hbm,
             src_v, dst_v, w_v, rows_v, neigh_sh, sem):
        cid = lax.axis_index("c")
        sid = lax.axis_index("s")
        start = jnp.where(cid == 0, sid * Q0, NS * Q0 + sid * Q1)
        nph = jnp.where(cid == 0, Q0 // PC, Q1 // PC)

        zeros = jnp.zeros((LANES,), jnp.float32)

        def zrow(i, _):
            for k in range(D // LANES):
                rows_v[i, pl.ds(k * LANES, LANES)] = zeros
            return 0

        lax.fori_loop(0, C, zrow, 0)
        base = sid * rpt
        for k in range(n_full):
            pltpu.sync_copy(rows_v, neigh_sh.at[pl.ds(base + k * C, C)])
        if tail:
            pltpu.sync_copy(rows_v.at[pl.ds(0, tail)],
                            neigh_sh.at[pl.ds(base + n_full * C, tail)])
        plsc.subcore_barrier()

        def phase(p, _):
            off = start + p * PC
            pltpu.sync_copy(src_hbm.at[pl.ds(off, PC)], src_v)
            pltpu.sync_copy(dst_hbm.at[pl.ds(off, PC)], dst_v)
            pltpu.sync_copy(w_hbm.at[pl.ds(off, PC)], w_v)
            for jj in range(PC):
                pltpu.async_copy(
                    nfeat_hbm.at[src_v.at[jj]], rows_v, sem).wait()

                def group(g, _):
                    wv = w_v[jj, pl.ds(g * LANES, LANES)]
                    for l in range(LANES):
                        ws = wv[l]
                        i = g * LANES + l
                        for k in range(D // LANES):
                            sl = pl.ds(k * LANES, LANES)
                            rows_v[i, sl] = rows_v[i, sl] * ws
                    return 0

                lax.fori_loop(0, C // LANES, group, 0)
                pltpu.sync_copy(rows_v, neigh_sh.at[dst_v.at[jj]], add=True)
            return 0

        lax.fori_loop(0, nph, phase, 0)

        plsc.subcore_barrier()
        pltpu.sync_copy(neigh_sh.at[pl.ds(base, rpt)],
                        out_hbm.at[cid, pl.ds(base, rpt)])

    run = pl.kernel(
        body,
        out_type=jax.ShapeDtypeStruct((NC, Npad, D), jnp.float32),
        mesh=mesh,
        scratch_types=[
            pltpu.VMEM((PC, C), jnp.int32),
            pltpu.VMEM((PC, C), jnp.int32),
            pltpu.VMEM((PC, C), jnp.float32),
            pltpu.VMEM((C, D), jnp.float32),
            pltpu.VMEM_SHARED((Npad, D), jnp.float32),
            pltpu.SemaphoreType.DMA,
        ],
    )
    return run(n_feat, src_r, dst_r, w_r)


def _tc_mlp(n_feat, partials, W1, b1, W2, b2):
    N, D = n_feat.shape
    BLK = 400
    grid = N // BLK

    def body(nf_ref, pp_ref, w1_ref, b1_ref, w2_ref, b2_ref, out_ref):
        rst = nf_ref[...] + pp_ref[0] + pp_ref[1]
        h = jnp.dot(rst, w1_ref[...], preferred_element_type=jnp.float32)
        h = jnp.maximum(h + b1_ref[...], 0.0)
        o = jnp.dot(h, w2_ref[...], preferred_element_type=jnp.float32)
        out_ref[...] = o + b2_ref[...]

    return pl.pallas_call(
        body,
        grid=(grid,),
        in_specs=[
            pl.BlockSpec((BLK, D), lambda i: (i, 0)),
            pl.BlockSpec((NC, BLK, D), lambda i: (0, i, 0)),
            pl.BlockSpec((D, D), lambda i: (0, 0)),
            pl.BlockSpec((1, D), lambda i: (0, 0)),
            pl.BlockSpec((D, D), lambda i: (0, 0)),
            pl.BlockSpec((1, D), lambda i: (0, 0)),
        ],
        out_specs=pl.BlockSpec((BLK, D), lambda i: (i, 0)),
        out_shape=jax.ShapeDtypeStruct((N, D), jnp.float32),
    )(n_feat, partials, W1, b1.reshape(1, D), W2, b2.reshape(1, D))


@jax.jit
def kernel(n_feat, e_feat, edge_weight, edge_index, W1, b1, W2, b2):
    del e_feat
    N, D = n_feat.shape
    E = edge_index.shape[1]
    totch = NS * (Q0 + Q1)
    E_pad = totch * C
    pad = E_pad - E
    assert pad >= 0

    src = edge_index[0].astype(jnp.int32)
    dst = edge_index[1].astype(jnp.int32)
    w = edge_weight[:, 0].astype(jnp.float32)
    if pad:
        src = jnp.concatenate([src, jnp.zeros((pad,), jnp.int32)])
        dst = jnp.concatenate([dst, jnp.zeros((pad,), jnp.int32)])
        w = jnp.concatenate([w, jnp.zeros((pad,), jnp.float32)])

    src_r = src.reshape(totch, C)
    dst_r = dst.reshape(totch, C)
    w_r = w.reshape(totch, C)

    partials = _sc_segment_sum(n_feat, src_r, dst_r, w_r)
    return _tc_mlp(n_feat, partials, W1, b1, W2, b2)


# speedup vs baseline: 1.5161x; 1.5161x over previous
import jax
import jax.numpy as jnp
from jax import lax
from jax.experimental import pallas as pl
from jax.experimental.pallas import tpu as pltpu
from jax.experimental.pallas import tpu_sc as plsc

NC = 2
NS = 16
NW = NC * NS
LANES = 16
C = 128


def _sc_segment_sum(n_feat2, N, src_r, dst_r, w_r, n_chunks):
    D = n_feat2.shape[1]
    rpt = -(-N // (NS * 8)) * 8
    Npad = rpt * NS
    n_full = rpt // C
    tail = rpt - n_full * C
    mesh = plsc.VectorSubcoreMesh(
        core_axis_name="c", subcore_axis_name="s",
        num_cores=NC, num_subcores=NS)

    def body(nfeat_hbm, src_hbm, dst_hbm, w_hbm, out_hbm,
             src_v, dst_v, w_v, rows_v, neigh_sh, sem):
        cid = lax.axis_index("c")
        sid = lax.axis_index("s")
        wid = sid * NC + cid

        pltpu.sync_copy(src_hbm.at[wid], src_v)
        pltpu.sync_copy(dst_hbm.at[wid], dst_v)
        pltpu.sync_copy(w_hbm.at[wid], w_v)

        off = cid * N

        def trow(r, _):
            for c8 in range(C // LANES):
                sl = pl.ds(c8 * LANES, LANES)
                src_v[r, sl] = src_v[r, sl] + off
            return 0

        lax.fori_loop(0, n_chunks, trow, 0)

        zeros = jnp.zeros((LANES,), jnp.float32)

        def zrow(i, _):
            for k in range(D // LANES):
                rows_v[i, pl.ds(k * LANES, LANES)] = zeros
            return 0

        lax.fori_loop(0, C, zrow, 0)
        base = sid * rpt
        for k in range(n_full):
            pltpu.sync_copy(rows_v, neigh_sh.at[pl.ds(base + k * C, C)])
        if tail:
            pltpu.sync_copy(rows_v.at[pl.ds(0, tail)],
                            neigh_sh.at[pl.ds(base + n_full * C, tail)])
        plsc.subcore_barrier()

        def step(j, _):
            pltpu.async_copy(nfeat_hbm.at[src_v.at[j]], rows_v, sem).wait()

            def group(g, _):
                wv = w_v[j, pl.ds(g * LANES, LANES)]
                for l in range(LANES):
                    ws = wv[l]
                    i = g * LANES + l
                    for k in range(D // LANES):
                        sl = pl.ds(k * LANES, LANES)
                        rows_v[i, sl] = rows_v[i, sl] * ws
                return 0

            lax.fori_loop(0, C // LANES, group, 0)
            pltpu.sync_copy(rows_v, neigh_sh.at[dst_v.at[j]], add=True)
            return 0

        lax.fori_loop(0, n_chunks, step, 0)

        plsc.subcore_barrier()
        pltpu.sync_copy(neigh_sh.at[pl.ds(base, rpt)],
                        out_hbm.at[cid, pl.ds(base, rpt)])

    run = pl.kernel(
        body,
        out_type=jax.ShapeDtypeStruct((NC, Npad, D), jnp.float32),
        mesh=mesh,
        scratch_types=[
            pltpu.VMEM((n_chunks, C), jnp.int32),
            pltpu.VMEM((n_chunks, C), jnp.int32),
            pltpu.VMEM((n_chunks, C), jnp.float32),
            pltpu.VMEM((C, D), jnp.float32),
            pltpu.VMEM_SHARED((Npad, D), jnp.float32),
            pltpu.SemaphoreType.DMA,
        ],
    )
    return run(n_feat2, src_r, dst_r, w_r)


def _tc_mlp(n_feat, partials, W1, b1, W2, b2):
    N, D = n_feat.shape
    BLK = 400
    grid = N // BLK

    def body(nf_ref, pp_ref, w1_ref, b1_ref, w2_ref, b2_ref, out_ref):
        rst = nf_ref[...] + pp_ref[0] + pp_ref[1]
        h = jnp.dot(rst, w1_ref[...], preferred_element_type=jnp.float32)
        h = jnp.maximum(h + b1_ref[...], 0.0)
        o = jnp.dot(h, w2_ref[...], preferred_element_type=jnp.float32)
        out_ref[...] = o + b2_ref[...]

    return pl.pallas_call(
        body,
        grid=(grid,),
        in_specs=[
            pl.BlockSpec((BLK, D), lambda i: (i, 0)),
            pl.BlockSpec((NC, BLK, D), lambda i: (0, i, 0)),
            pl.BlockSpec((D, D), lambda i: (0, 0)),
            pl.BlockSpec((1, D), lambda i: (0, 0)),
            pl.BlockSpec((D, D), lambda i: (0, 0)),
            pl.BlockSpec((1, D), lambda i: (0, 0)),
        ],
        out_specs=pl.BlockSpec((BLK, D), lambda i: (i, 0)),
        out_shape=jax.ShapeDtypeStruct((N, D), jnp.float32),
    )(n_feat, partials, W1, b1.reshape(1, D), W2, b2.reshape(1, D))


@jax.jit
def kernel(n_feat, e_feat, edge_weight, edge_index, W1, b1, W2, b2):
    del e_feat
    N, D = n_feat.shape
    E = edge_index.shape[1]
    epw = -(-E // NW)
    epw = -(-epw // C) * C
    E_pad = epw * NW
    pad = E_pad - E

    src = edge_index[0].astype(jnp.int32)
    dst = edge_index[1].astype(jnp.int32)
    w = edge_weight[:, 0].astype(jnp.float32)
    if pad:
        src = jnp.concatenate([src, jnp.zeros((pad,), jnp.int32)])
        dst = jnp.concatenate([dst, jnp.zeros((pad,), jnp.int32)])
        w = jnp.concatenate([w, jnp.zeros((pad,), jnp.float32)])

    n_chunks = epw // C
    src_r = src.reshape(NW, n_chunks, C)
    dst_r = dst.reshape(NW, n_chunks, C)
    w_r = w.reshape(NW, n_chunks, C)

    n_feat2 = jnp.concatenate([n_feat, n_feat], axis=0)
    partials = _sc_segment_sum(n_feat2, N, src_r, dst_r, w_r, n_chunks)
    return _tc_mlp(n_feat, partials, W1, b1, W2, b2)


# speedup vs baseline: 1.7015x; 1.1223x over previous
import jax
import jax.numpy as jnp
from jax import lax
from jax.experimental import pallas as pl
from jax.experimental.pallas import tpu as pltpu
from jax.experimental.pallas import tpu_sc as plsc

NC = 2
NS = 16
NW = NC * NS
LANES = 16
C = 128


def _sc_segment_sum(n_feat, sd_r, w_r, n_chunks):
    N, D = n_feat.shape
    rpt = -(-N // (NS * 8)) * 8
    Npad = rpt * NS
    n_full = rpt // C
    tail = rpt - n_full * C
    mesh = plsc.VectorSubcoreMesh(
        core_axis_name="c", subcore_axis_name="s",
        num_cores=NC, num_subcores=NS)

    def body(nfeat_hbm, sd_hbm, w_hbm, out_hbm,
             src_v, dst_v, w_v, rows_v, neigh_sh, sem):
        cid = lax.axis_index("c")
        sid = lax.axis_index("s")
        wid = sid * NC + cid

        pltpu.sync_copy(sd_hbm.at[0, wid], src_v)
        pltpu.sync_copy(sd_hbm.at[1, wid], dst_v)
        pltpu.sync_copy(w_hbm.at[wid], w_v)

        zeros = jnp.zeros((LANES,), jnp.float32)

        def zrow(i, _):
            for k in range(D // LANES):
                rows_v[i, pl.ds(k * LANES, LANES)] = zeros
            return 0

        lax.fori_loop(0, C, zrow, 0)
        base = sid * rpt
        for k in range(n_full):
            pltpu.sync_copy(rows_v, neigh_sh.at[pl.ds(base + k * C, C)])
        if tail:
            pltpu.sync_copy(rows_v.at[pl.ds(0, tail)],
                            neigh_sh.at[pl.ds(base + n_full * C, tail)])
        plsc.subcore_barrier()

        def step(j, _):
            pltpu.async_copy(nfeat_hbm.at[src_v.at[j]], rows_v, sem).wait()

            def group(g, _):
                wv = w_v[j, pl.ds(g * LANES, LANES)]
                for l in range(LANES):
                    ws = wv[l]
                    i = g * LANES + l
                    for k in range(D // LANES):
                        sl = pl.ds(k * LANES, LANES)
                        rows_v[i, sl] = rows_v[i, sl] * ws
                return 0

            lax.fori_loop(0, C // LANES, group, 0)
            pltpu.sync_copy(rows_v, neigh_sh.at[dst_v.at[j]], add=True)
            return 0

        lax.fori_loop(0, n_chunks, step, 0)

        plsc.subcore_barrier()
        pltpu.sync_copy(neigh_sh.at[pl.ds(base, rpt)],
                        out_hbm.at[cid, pl.ds(base, rpt)])

    run = pl.kernel(
        body,
        out_type=jax.ShapeDtypeStruct((NC, Npad, D), jnp.float32),
        mesh=mesh,
        scratch_types=[
            pltpu.VMEM((n_chunks, C), jnp.int32),
            pltpu.VMEM((n_chunks, C), jnp.int32),
            pltpu.VMEM((n_chunks, C), jnp.float32),
            pltpu.VMEM((C, D), jnp.float32),
            pltpu.VMEM_SHARED((Npad, D), jnp.float32),
            pltpu.SemaphoreType.DMA,
        ],
    )
    return run(n_feat, sd_r, w_r)


def _tc_mlp(n_feat, partials, W1, b1, W2, b2):
    N, D = n_feat.shape
    BLK = 2000
    grid = N // BLK

    def body(nf_ref, pp_ref, w1_ref, b1_ref, w2_ref, b2_ref, out_ref):
        rst = nf_ref[...] + pp_ref[0] + pp_ref[1]
        h = jnp.dot(rst, w1_ref[...], preferred_element_type=jnp.float32)
        h = jnp.maximum(h + b1_ref[...], 0.0)
        o = jnp.dot(h, w2_ref[...], preferred_element_type=jnp.float32)
        out_ref[...] = o + b2_ref[...]

    return pl.pallas_call(
        body,
        grid=(grid,),
        in_specs=[
            pl.BlockSpec((BLK, D), lambda i: (i, 0)),
            pl.BlockSpec((NC, BLK, D), lambda i: (0, i, 0)),
            pl.BlockSpec((D, D), lambda i: (0, 0)),
            pl.BlockSpec((1, D), lambda i: (0, 0)),
            pl.BlockSpec((D, D), lambda i: (0, 0)),
            pl.BlockSpec((1, D), lambda i: (0, 0)),
        ],
        out_specs=pl.BlockSpec((BLK, D), lambda i: (i, 0)),
        out_shape=jax.ShapeDtypeStruct((N, D), jnp.float32),
    )(n_feat, partials, W1, b1.reshape(1, D), W2, b2.reshape(1, D))


@jax.jit
def kernel(n_feat, e_feat, edge_weight, edge_index, W1, b1, W2, b2):
    del e_feat
    N, D = n_feat.shape
    E = edge_index.shape[1]
    epw = -(-E // NW)
    epw = -(-epw // C) * C
    E_pad = epw * NW
    pad = E_pad - E

    sd = edge_index.astype(jnp.int32)
    w = edge_weight[:, 0].astype(jnp.float32)
    if pad:
        sd = jnp.concatenate([sd, jnp.zeros((2, pad), jnp.int32)], axis=1)
        w = jnp.concatenate([w, jnp.zeros((pad,), jnp.float32)])

    n_chunks = epw // C
    sd_r = sd.reshape(2, NW, n_chunks, C)
    w_r = w.reshape(NW, n_chunks, C)

    partials = _sc_segment_sum(n_feat, sd_r, w_r, n_chunks)
    return _tc_mlp(n_feat, partials, W1, b1, W2, b2)
